# Initial kernel scaffold; baseline (speedup 1.0000x reference)
import jax
import jax.numpy as jnp
from jax.experimental import pallas as pl


def kernel(x, expert_indices, expert_gate, num_tokens, W1, b1, W2, b2):
    raise NotImplementedError("write your pallas kernel here")


# speedup vs baseline: 1.6662x; 1.6662x over previous
import functools

import jax
import jax.numpy as jnp
from jax import lax
from jax.experimental import pallas as pl
from jax.experimental.pallas import tpu as pltpu
from jax.experimental.pallas import tpu_sc as plsc

_B, _T, _D = 4, 2048, 2048
_E, _C = 16, 256
_O = 2048
_Oe = _O // _E
_Ie = _D // _E
_EC = _E * _C

_NC, _NS = 2, 16
_NW = _NC * _NS
_ROWS = _B * _EC
_RPW = _ROWS // _NW
_GCH = 32
_NGCH = _RPW // _GCH

_BPC = _B // _NC
_FW = 32
_NP = _O // (_NS * _FW)
_FCH = 8


def _gather_body(x_hbm, idx_hbm, sel_hbm, idx_v, rows_v, sem):
    c = lax.axis_index("c")
    s = lax.axis_index("s")
    base = (s * _NC + c) * _RPW

    def body(i, carry):
        off = base + i * _GCH
        pltpu.sync_copy(idx_hbm.at[pl.ds(off, _GCH)], idx_v)
        pltpu.async_copy(x_hbm.at[idx_v], rows_v, sem).wait()
        pltpu.sync_copy(rows_v, sel_hbm.at[pl.ds(off, _GCH)])
        return carry

    lax.fori_loop(0, _NGCH, body, 0)


@functools.cache
def _sc_kernels():
    mesh = plsc.VectorSubcoreMesh(
        core_axis_name="c", subcore_axis_name="s",
        num_cores=_NC, num_subcores=_NS)
    gather = pl.kernel(
        _gather_body,
        out_type=jax.ShapeDtypeStruct((_ROWS, _D), jnp.float32),
        mesh=mesh,
        scratch_types=[
            pltpu.VMEM((_GCH,), jnp.int32),
            pltpu.VMEM((_GCH, _D), jnp.float32),
            pltpu.SemaphoreType.DMA,
        ],
    )
    return (gather,)


def _mlp_body(sel_ref, w1_ref, b1_ref, w2_ref, b2_ref, g_ref, out_ref):
    xs = sel_ref[0, 0]
    w1 = w1_ref[0]
    h = lax.dot_general(xs, w1, (((1,), (1,)), ((), ())),
                        preferred_element_type=jnp.float32)
    h = h + b1_ref[0]
    h = 0.5 * h * (1.0 + lax.erf(h * 0.7071067811865476))
    w2 = w2_ref[0]
    y = lax.dot_general(h, w2, (((1,), (1,)), ((), ())),
                        preferred_element_type=jnp.float32)
    y = y + b2_ref[...]
    y = y * jnp.reshape(g_ref[0, 0], (_C, 1))
    out_ref[0, 0] = y


def _dense_mlp(sel4, w1e, b1e, w2e, b2r, g4):
    return pl.pallas_call(
        _mlp_body,
        grid=(_E, _B),
        in_specs=[
            pl.BlockSpec((1, 1, _C, _D), lambda e, b: (b, e, 0, 0)),
            pl.BlockSpec((1, _Oe, _D), lambda e, b: (e, 0, 0)),
            pl.BlockSpec((1, 1, _Oe), lambda e, b: (e, 0, 0)),
            pl.BlockSpec((1, _O, _Ie), lambda e, b: (e, 0, 0)),
            pl.BlockSpec((1, _O), lambda e, b: (0, 0)),
            pl.BlockSpec((1, 1, 1, _C), lambda e, b: (b, e, 0, 0)),
        ],
        out_specs=pl.BlockSpec((1, 1, _C, _O), lambda e, b: (b, e, 0, 0)),
        out_shape=jax.ShapeDtypeStruct((_B, _E, _C, _O), jnp.float32),
        compiler_params=pltpu.CompilerParams(
            dimension_semantics=("arbitrary", "arbitrary")),
    )(sel4, w1e, b1e, w2e, b2r, g4)


_RCH = 512
_NKC = _EC // _RCH


def _tc_scatter_body(idx_s, h3_ref, out_ref):
    b = pl.program_id(0)
    k = pl.program_id(1)

    @pl.when(k == 0)
    def _zero():
        out_ref[...] = jnp.zeros_like(out_ref)

    def body(i, carry):
        r = idx_s[b * _EC + k * _RCH + i]
        src = h3_ref[0, pl.ds(i * 16, 16), :]
        off = pl.multiple_of(r * 16, 16)
        out_ref[0, pl.ds(off, 16), :] += src
        return carry

    lax.fori_loop(0, _RCH, body, 0)


def _tc_scatter(h3, idx_flat_raw):
    return pl.pallas_call(
        _tc_scatter_body,
        grid_spec=pltpu.PrefetchScalarGridSpec(
            num_scalar_prefetch=1,
            grid=(_B, _NKC),
            in_specs=[
                pl.BlockSpec((1, _RCH * 16, 128), lambda b, k, idx: (b, k, 0)),
            ],
            out_specs=pl.BlockSpec((1, _T * 16, 128), lambda b, k, idx: (b, 0, 0)),
        ),
        out_shape=jax.ShapeDtypeStruct((_B, _T * 16, 128), jnp.float32),
        compiler_params=pltpu.CompilerParams(
            dimension_semantics=("arbitrary", "arbitrary")),
    )(idx_flat_raw, h3)


def kernel(x, expert_indices, expert_gate, num_tokens, W1, b1, W2, b2):
    x_flat = x.reshape(_B * _T, _D)
    idx_raw = expert_indices.reshape(-1)
    offs = jnp.repeat(jnp.arange(_B, dtype=jnp.int32) * _T, _EC)
    idx_flat = idx_raw + offs

    (gather_rows,) = _sc_kernels()
    sel = gather_rows(x_flat, idx_flat)

    sel4 = sel.reshape(_B, _E, _C, _D)
    w1e = W1.reshape(_E, _Oe, _D)
    b1e = b1.reshape(_E, 1, _Oe)
    w2e = W2.reshape(_E, _O, _Ie)
    b2r = b2.reshape(1, _O)
    g4 = expert_gate.reshape(_B, _E, 1, _C)
    h2 = _dense_mlp(sel4, w1e, b1e, w2e, b2r, g4)

    h3 = h2.reshape(_B, _EC * 16, 128)
    out3 = _tc_scatter(h3, idx_raw)
    return out3.reshape(_B, _T, _O)


# speedup vs baseline: 1.6946x; 1.0170x over previous
import functools

import jax
import jax.numpy as jnp
from jax import lax
from jax.experimental import pallas as pl
from jax.experimental.pallas import tpu as pltpu
from jax.experimental.pallas import tpu_sc as plsc

_B, _T, _D = 4, 2048, 2048
_E, _C = 16, 256
_O = 2048
_Oe = _O // _E
_Ie = _D // _E
_EC = _E * _C

_NC, _NS = 2, 16
_NW = _NC * _NS
_ROWS = _B * _EC
_RPW = _ROWS // _NW
_GCH = 16
_NGCH = _RPW // _GCH

_BPC = _B // _NC
_FW = 32
_NP = _O // (_NS * _FW)
_FCH = 8


def _gather_body(x_hbm, idx_hbm, sel_hbm,
                 idx0, idx1, rows0, rows1, sg0, sg1, sw0, sw1):
    c = lax.axis_index("c")
    s = lax.axis_index("s")
    base = (s * _NC + c) * _RPW
    idxb, rowsb, sgb, swb = (idx0, idx1), (rows0, rows1), (sg0, sg1), (sw0, sw1)

    gd = [None] * _NGCH
    wd = [None] * _NGCH
    for i in range(_NGCH):
        bi = i % 2
        if i >= 2:
            wd[i - 2].wait()
        off = base + i * _GCH
        pltpu.sync_copy(idx_hbm.at[pl.ds(off, _GCH)], idxb[bi])
        gd[i] = pltpu.async_copy(x_hbm.at[idxb[bi]], rowsb[bi], sgb[bi])
        if i >= 1:
            gd[i - 1].wait()
            offp = base + (i - 1) * _GCH
            wd[i - 1] = pltpu.async_copy(
                rowsb[(i - 1) % 2], sel_hbm.at[pl.ds(offp, _GCH)],
                swb[(i - 1) % 2])
    last = _NGCH - 1
    gd[last].wait()
    wd[last] = pltpu.async_copy(
        rowsb[last % 2], sel_hbm.at[pl.ds(base + last * _GCH, _GCH)],
        swb[last % 2])
    wd[last - 1].wait()
    wd[last].wait()


@functools.cache
def _sc_kernels():
    mesh = plsc.VectorSubcoreMesh(
        core_axis_name="c", subcore_axis_name="s",
        num_cores=_NC, num_subcores=_NS)
    gather = pl.kernel(
        _gather_body,
        out_type=jax.ShapeDtypeStruct((_ROWS, _D), jnp.float32),
        mesh=mesh,
        scratch_types=[
            pltpu.VMEM((_GCH,), jnp.int32),
            pltpu.VMEM((_GCH,), jnp.int32),
            pltpu.VMEM((_GCH, _D), jnp.float32),
            pltpu.VMEM((_GCH, _D), jnp.float32),
            pltpu.SemaphoreType.DMA,
            pltpu.SemaphoreType.DMA,
            pltpu.SemaphoreType.DMA,
            pltpu.SemaphoreType.DMA,
        ],
    )
    return (gather,)


def _mlp_body(sel_ref, w1_ref, b1_ref, w2_ref, b2_ref, g_ref, out_ref):
    xs = sel_ref[0, 0]
    w1 = w1_ref[0]
    h = lax.dot_general(xs, w1, (((1,), (1,)), ((), ())),
                        preferred_element_type=jnp.float32)
    h = h + b1_ref[0]
    h = 0.5 * h * (1.0 + lax.erf(h * 0.7071067811865476))
    w2 = w2_ref[0]
    y = lax.dot_general(h, w2, (((1,), (1,)), ((), ())),
                        preferred_element_type=jnp.float32)
    y = y + b2_ref[...]
    y = y * jnp.reshape(g_ref[0, 0], (_C, 1))
    out_ref[0, 0] = y


def _dense_mlp(sel4, w1e, b1e, w2e, b2r, g4):
    return pl.pallas_call(
        _mlp_body,
        grid=(_E, _B),
        in_specs=[
            pl.BlockSpec((1, 1, _C, _D), lambda e, b: (b, e, 0, 0)),
            pl.BlockSpec((1, _Oe, _D), lambda e, b: (e, 0, 0)),
            pl.BlockSpec((1, 1, _Oe), lambda e, b: (e, 0, 0)),
            pl.BlockSpec((1, _O, _Ie), lambda e, b: (e, 0, 0)),
            pl.BlockSpec((1, _O), lambda e, b: (0, 0)),
            pl.BlockSpec((1, 1, 1, _C), lambda e, b: (b, e, 0, 0)),
        ],
        out_specs=pl.BlockSpec((1, 1, _C, _O), lambda e, b: (b, e, 0, 0)),
        out_shape=jax.ShapeDtypeStruct((_B, _E, _C, _O), jnp.float32),
        compiler_params=pltpu.CompilerParams(
            dimension_semantics=("arbitrary", "arbitrary")),
    )(sel4, w1e, b1e, w2e, b2r, g4)


_RCH = 512
_NKC = _EC // _RCH


def _tc_scatter_body(idx_s, h3_ref, out_ref):
    b = pl.program_id(0)
    k = pl.program_id(1)

    @pl.when(k == 0)
    def _zero():
        out_ref[...] = jnp.zeros_like(out_ref)

    def body(i, carry):
        r = idx_s[b * _EC + k * _RCH + i]
        src = h3_ref[0, pl.ds(i * 16, 16), :]
        off = pl.multiple_of(r * 16, 16)
        out_ref[0, pl.ds(off, 16), :] += src
        return carry

    lax.fori_loop(0, _RCH, body, 0)


def _tc_scatter(h3, idx_flat_raw):
    return pl.pallas_call(
        _tc_scatter_body,
        grid_spec=pltpu.PrefetchScalarGridSpec(
            num_scalar_prefetch=1,
            grid=(_B, _NKC),
            in_specs=[
                pl.BlockSpec((1, _RCH * 16, 128), lambda b, k, idx: (b, k, 0)),
            ],
            out_specs=pl.BlockSpec((1, _T * 16, 128), lambda b, k, idx: (b, 0, 0)),
        ),
        out_shape=jax.ShapeDtypeStruct((_B, _T * 16, 128), jnp.float32),
        compiler_params=pltpu.CompilerParams(
            dimension_semantics=("arbitrary", "arbitrary")),
    )(idx_flat_raw, h3)


def kernel(x, expert_indices, expert_gate, num_tokens, W1, b1, W2, b2):
    x_flat = x.reshape(_B * _T, _D)
    idx_raw = expert_indices.reshape(-1)
    offs = jnp.repeat(jnp.arange(_B, dtype=jnp.int32) * _T, _EC)
    idx_flat = idx_raw + offs

    (gather_rows,) = _sc_kernels()
    sel = gather_rows(x_flat, idx_flat)

    sel4 = sel.reshape(_B, _E, _C, _D)
    w1e = W1.reshape(_E, _Oe, _D)
    b1e = b1.reshape(_E, 1, _Oe)
    w2e = W2.reshape(_E, _O, _Ie)
    b2r = b2.reshape(1, _O)
    g4 = expert_gate.reshape(_B, _E, 1, _C)
    h2 = _dense_mlp(sel4, w1e, b1e, w2e, b2r, g4)

    h3 = h2.reshape(_B, _EC * 16, 128)
    out3 = _tc_scatter(h3, idx_raw)
    return out3.reshape(_B, _T, _O)
